# baseline (speedup 1.0000x reference)
import functools

import jax
import jax.numpy as jnp
from jax import lax
from jax.experimental import pallas as pl
from jax.experimental.pallas import tpu as pltpu
from jax.experimental.pallas import tpu_sc as plsc

B, C, H, W = 4, 96, 224, 224
CROP_H, CROP_W = 14, 14
N_BOX = 1024
NPIX = N_BOX * CROP_H * CROP_W
HW = H * W

NC, NS, LANES = 2, 16, 16
NW = NC * NS
ROWS_PER_W = NPIX // NW
CHUNK = 112
NCHUNK = ROWS_PER_W // CHUNK
NPACK = C // 2
TABLE_D = 128


def _index_body(boxes_ref, bind_ref, ptl_ref, ptr_ref,
                wtl_ref, wtr_ref, wbl_ref, wbr_ref):
    f32 = jnp.float32
    boxes = boxes_ref[...]
    bind = bind_ref[...]
    y1 = boxes[:, 0:1]
    x1 = boxes[:, 1:2]
    y2 = boxes[:, 2:3]
    x2 = boxes[:, 3:4]
    col = lax.broadcasted_iota(jnp.int32, (N_BOX, CROP_H * CROP_W), 1)
    i_f = (col // CROP_W).astype(f32)
    j_f = (col % CROP_W).astype(f32)
    height_scale = (y2 - y1) * (H - 1) / (CROP_H - 1)
    width_scale = (x2 - x1) * (W - 1) / (CROP_W - 1)
    in_y = y1 * (H - 1) + i_f * height_scale
    in_x = x1 * (W - 1) + j_f * width_scale
    ti = in_y.astype(jnp.int32)
    li = in_x.astype(jnp.int32)
    yl = in_y - ti.astype(f32)
    xl = in_x - li.astype(f32)
    p_tl = bind * HW + ti * W + li
    ptl_ref[...] = p_tl
    ptr_ref[...] = p_tl + W
    one = f32(1.0)
    wtl_ref[...] = (one - yl) * (one - xl)
    wtr_ref[...] = (one - yl) * xl
    wbl_ref[...] = yl * (one - xl)
    wbr_ref[...] = yl * xl


BB = 32


YB = 16


def _in_transpose_body(img_ref, tab_ref):
    x = img_ref[0].reshape(C, YB * W)
    bits = lax.bitcast_convert_type(
        x.astype(jnp.bfloat16).astype(jnp.float32), jnp.int32)
    lo = lax.shift_right_logical(bits, 16)
    hi = bits & jnp.int32(-65536)
    hi_r = jnp.concatenate([hi[16:, :], hi[:16, :]], axis=0)
    y = lo | hi_r
    packed = jnp.concatenate([y[0:16], y[32:48], y[64:80]], axis=0)
    t = jnp.transpose(packed, (1, 0))
    t1 = jnp.concatenate([t[1:], t[:1]], axis=0)
    pad = jnp.zeros((YB * W, TABLE_D - 2 * NPACK), jnp.int32)
    tab_ref[...] = jnp.concatenate([t, t1, pad], axis=1)


def _make_table(image):
    nyb = H // YB
    return pl.pallas_call(
        _in_transpose_body,
        grid=(B, nyb),
        in_specs=[pl.BlockSpec((1, C, YB, W), lambda b, y: (b, 0, y, 0))],
        out_specs=pl.BlockSpec((YB * W, TABLE_D),
                               lambda b, y: (b * nyb + y, 0)),
        out_shape=jax.ShapeDtypeStruct((B * HW, TABLE_D), jnp.int32),
    )(image)


def _out_transpose_body(rows_ref, out_ref):
    x = rows_ref[...].reshape(BB, CROP_H * CROP_W, C)
    out_ref[...] = jnp.transpose(x, (0, 2, 1))


def _to_nchw(out_t):
    npb = CROP_H * CROP_W
    return pl.pallas_call(
        _out_transpose_body,
        grid=(N_BOX // BB,),
        in_specs=[pl.BlockSpec((BB * npb, C), lambda i: (i, 0))],
        out_specs=pl.BlockSpec((BB, C, npb), lambda i: (i, 0, 0)),
        out_shape=jax.ShapeDtypeStruct((N_BOX, C, npb), jnp.float32),
    )(out_t)


def _compute_indices(boxes, box_ind):
    shp = jax.ShapeDtypeStruct((N_BOX, CROP_H * CROP_W), jnp.int32)
    shpf = jax.ShapeDtypeStruct((N_BOX, CROP_H * CROP_W), jnp.float32)
    return pl.pallas_call(
        _index_body,
        out_shape=(shp, shp, shpf, shpf, shpf, shpf),
    )(boxes, box_ind.reshape(N_BOX, 1))


def _sc_body(table, ptl, ptr, wtl, wtr, wbl, wbr, out,
             itl, itr, vtl, vtr, vbl, vbr,
             bufs0, bufs1, rout0, rout1, sem0, sem1, sem_out):
    wid = lax.axis_index("s") * NC + lax.axis_index("c")
    base0 = pl.multiple_of(wid * ROWS_PER_W, CHUNK)
    span = pl.ds(base0, ROWS_PER_W)

    pltpu.sync_copy(ptl.at[span], itl)
    pltpu.sync_copy(ptr.at[span], itr)
    pltpu.sync_copy(wtl.at[span], vtl)
    pltpu.sync_copy(wtr.at[span], vtr)
    pltpu.sync_copy(wbl.at[span], vbl)
    pltpu.sync_copy(wbr.at[span], vbr)

    ring = ((bufs0, sem0), (bufs1, sem1))

    def fire(ci, bufs, sem):
        off = pl.multiple_of(ci * CHUNK, CHUNK)
        pltpu.async_copy(table.at[itl.at[pl.ds(off, CHUNK)]], bufs.at[0], sem)
        pltpu.async_copy(table.at[itr.at[pl.ds(off, CHUNK)]], bufs.at[1], sem)

    def drain(bufs, sem):
        for k in range(2):
            pltpu.make_async_copy(table.at[pl.ds(0, CHUNK)],
                                  bufs.at[k], sem).wait()

    def blend(ci, bufs, rout):
        off = pl.multiple_of(ci * CHUNK, CHUNK)

        def group_body(g, gcarry):
            r0 = pl.multiple_of(g * LANES, LANES)
            wa = vtl[pl.ds(off + r0, LANES)]
            wb = vtr[pl.ds(off + r0, LANES)]
            wc = vbl[pl.ds(off + r0, LANES)]
            wd = vbr[pl.ds(off + r0, LANES)]
            mask_hi = jnp.int32(-65536)
            for k in range(LANES):
                r = r0 + k
                a, b, c, d = wa[k], wb[k], wc[k], wd[k]
                for g3 in range(C // 32):
                    xtl = bufs[0, r, pl.ds(g3 * LANES, LANES)]
                    xtr = bufs[0, r, pl.ds(NPACK + g3 * LANES, LANES)]
                    xbl = bufs[1, r, pl.ds(g3 * LANES, LANES)]
                    xbr = bufs[1, r, pl.ds(NPACK + g3 * LANES, LANES)]
                    acc_lo = (plsc.bitcast(xtl << 16, jnp.float32) * a
                              + plsc.bitcast(xtr << 16, jnp.float32) * b
                              + plsc.bitcast(xbl << 16, jnp.float32) * c
                              + plsc.bitcast(xbr << 16, jnp.float32) * d)
                    acc_hi = (plsc.bitcast(xtl & mask_hi, jnp.float32) * a
                              + plsc.bitcast(xtr & mask_hi, jnp.float32) * b
                              + plsc.bitcast(xbl & mask_hi, jnp.float32) * c
                              + plsc.bitcast(xbr & mask_hi, jnp.float32) * d)
                    rout[r, pl.ds(g3 * 32, LANES)] = acc_lo
                    rout[r, pl.ds(g3 * 32 + LANES, LANES)] = acc_hi
            return gcarry

        lax.fori_loop(0, CHUNK // LANES, group_body, 0, unroll=False)

    def out_fire(ci, rout):
        off = pl.multiple_of(ci * CHUNK, CHUNK)
        pltpu.async_copy(rout, out.at[pl.ds(base0 + off, CHUNK)], sem_out)

    def out_drain(rout):
        pltpu.make_async_copy(rout, out.at[pl.ds(base0, CHUNK)],
                              sem_out).wait()

    fire(0, *ring[0])

    def pair_body(g, carry):
        c0 = pl.multiple_of(g * 2, 2)
        fire(c0 + 1, *ring[1])
        drain(*ring[0])

        @pl.when(c0 > 0)
        def _():
            out_drain(rout0)

        blend(c0, ring[0][0], rout0)
        out_fire(c0, rout0)

        @pl.when(c0 + 2 < NCHUNK)
        def _():
            fire(c0 + 2, *ring[0])

        drain(*ring[1])

        @pl.when(c0 > 0)
        def _():
            out_drain(rout1)

        blend(c0 + 1, ring[1][0], rout1)
        out_fire(c0 + 1, rout1)
        return carry

    lax.fori_loop(0, NCHUNK // 2, pair_body, 0, unroll=False)
    out_drain(rout0)
    out_drain(rout1)


@functools.cache
def _make_sc_gather():
    return functools.partial(
        pl.kernel,
        out_type=jax.ShapeDtypeStruct((NPIX, C), jnp.float32),
        mesh=plsc.VectorSubcoreMesh(
            core_axis_name="c", subcore_axis_name="s",
            num_cores=NC, num_subcores=NS),
        scratch_types=(
            [pltpu.VMEM((ROWS_PER_W,), jnp.int32)] * 2
            + [pltpu.VMEM((ROWS_PER_W,), jnp.float32)] * 4
            + [pltpu.VMEM((2, CHUNK, TABLE_D), jnp.int32)] * 2
            + [pltpu.VMEM((CHUNK, C), jnp.float32)] * 2
            + [pltpu.SemaphoreType.DMA] * 3
        ),
        compiler_params=pltpu.CompilerParams(needs_layout_passes=False),
    )(_sc_body)


def kernel(image, boxes, box_ind):
    table = _make_table(image)
    ptl, ptr, wtl, wtr, wbl, wbr = _compute_indices(boxes, box_ind)
    flat = lambda a: a.reshape(NPIX)
    out_t = _make_sc_gather()(table, flat(ptl), flat(ptr), flat(wtl),
                              flat(wtr), flat(wbl), flat(wbr))
    return jnp.transpose(out_t.reshape(N_BOX, CROP_H, CROP_W, C),
                         (0, 3, 1, 2))


# speedup vs baseline: 1.0904x; 1.0904x over previous
import functools

import jax
import jax.numpy as jnp
from jax import lax
from jax.experimental import pallas as pl
from jax.experimental.pallas import tpu as pltpu
from jax.experimental.pallas import tpu_sc as plsc

B, C, H, W = 4, 96, 224, 224
CROP_H, CROP_W = 14, 14
N_BOX = 1024
NPIX = N_BOX * CROP_H * CROP_W
HW = H * W

NC, NS, LANES = 2, 16, 16
NW = NC * NS
ROWS_PER_W = NPIX // NW
CHUNK = 112
NCHUNK = ROWS_PER_W // CHUNK
NPACK = C // 2
TABLE_D = 128


def _index_body(boxes_ref, bind_ref, ptl_ref, ptr_ref,
                wtl_ref, wtr_ref, wbl_ref, wbr_ref):
    f32 = jnp.float32
    boxes = boxes_ref[...]
    bind = bind_ref[...]
    y1 = boxes[:, 0:1]
    x1 = boxes[:, 1:2]
    y2 = boxes[:, 2:3]
    x2 = boxes[:, 3:4]
    col = lax.broadcasted_iota(jnp.int32, (N_BOX, CROP_H * CROP_W), 1)
    i_f = (col // CROP_W).astype(f32)
    j_f = (col % CROP_W).astype(f32)
    height_scale = (y2 - y1) * (H - 1) / (CROP_H - 1)
    width_scale = (x2 - x1) * (W - 1) / (CROP_W - 1)
    in_y = y1 * (H - 1) + i_f * height_scale
    in_x = x1 * (W - 1) + j_f * width_scale
    ti = in_y.astype(jnp.int32)
    li = in_x.astype(jnp.int32)
    yl = in_y - ti.astype(f32)
    xl = in_x - li.astype(f32)
    p_tl = bind * HW + ti * W + li
    ptl_ref[...] = p_tl
    ptr_ref[...] = p_tl + W
    one = f32(1.0)
    wtl_ref[...] = (one - yl) * (one - xl)
    wtr_ref[...] = (one - yl) * xl
    wbl_ref[...] = yl * (one - xl)
    wbr_ref[...] = yl * xl


BB = 32


YB = 32


def _in_transpose_body(img_ref, tab_ref):
    x = img_ref[0].reshape(C, YB * W)
    bits = lax.bitcast_convert_type(
        x.astype(jnp.bfloat16).astype(jnp.float32), jnp.int32)
    lo = lax.shift_right_logical(bits, 16)
    hi = bits & jnp.int32(-65536)
    hi_r = jnp.concatenate([hi[16:, :], hi[:16, :]], axis=0)
    y = lo | hi_r
    packed = jnp.concatenate([y[0:16], y[32:48], y[64:80]], axis=0)
    t = jnp.transpose(packed, (1, 0))
    t1 = jnp.concatenate([t[1:], t[:1]], axis=0)
    pad = jnp.zeros((YB * W, TABLE_D - 2 * NPACK), jnp.int32)
    tab_ref[...] = jnp.concatenate([t, t1, pad], axis=1)


def _make_table(image):
    nyb = H // YB
    return pl.pallas_call(
        _in_transpose_body,
        grid=(B, nyb),
        in_specs=[pl.BlockSpec((1, C, YB, W), lambda b, y: (b, 0, y, 0))],
        out_specs=pl.BlockSpec((YB * W, TABLE_D),
                               lambda b, y: (b * nyb + y, 0)),
        out_shape=jax.ShapeDtypeStruct((B * HW, TABLE_D), jnp.int32),
    )(image)


def _out_transpose_body(rows_ref, out_ref):
    x = rows_ref[...].reshape(BB, CROP_H * CROP_W, C)
    out_ref[...] = jnp.transpose(x, (0, 2, 1))


def _to_nchw(out_t):
    npb = CROP_H * CROP_W
    return pl.pallas_call(
        _out_transpose_body,
        grid=(N_BOX // BB,),
        in_specs=[pl.BlockSpec((BB * npb, C), lambda i: (i, 0))],
        out_specs=pl.BlockSpec((BB, C, npb), lambda i: (i, 0, 0)),
        out_shape=jax.ShapeDtypeStruct((N_BOX, C, npb), jnp.float32),
    )(out_t)


def _compute_indices(boxes, box_ind):
    shp = jax.ShapeDtypeStruct((N_BOX, CROP_H * CROP_W), jnp.int32)
    shpf = jax.ShapeDtypeStruct((N_BOX, CROP_H * CROP_W), jnp.float32)
    return pl.pallas_call(
        _index_body,
        out_shape=(shp, shp, shpf, shpf, shpf, shpf),
    )(boxes, box_ind.reshape(N_BOX, 1))


def _sc_body(table, ptl, ptr, wtl, wtr, wbl, wbr, out,
             itl, itr, vtl, vtr, vbl, vbr,
             bufs0, bufs1, rout0, rout1, sem0, sem1, sem_out):
    wid = lax.axis_index("s") * NC + lax.axis_index("c")
    base0 = pl.multiple_of(wid * ROWS_PER_W, CHUNK)
    span = pl.ds(base0, ROWS_PER_W)

    pltpu.sync_copy(ptl.at[span], itl)
    pltpu.sync_copy(ptr.at[span], itr)
    pltpu.sync_copy(wtl.at[span], vtl)
    pltpu.sync_copy(wtr.at[span], vtr)
    pltpu.sync_copy(wbl.at[span], vbl)
    pltpu.sync_copy(wbr.at[span], vbr)

    ring = ((bufs0, sem0), (bufs1, sem1))

    def fire(ci, bufs, sem):
        off = pl.multiple_of(ci * CHUNK, CHUNK)
        pltpu.async_copy(table.at[itl.at[pl.ds(off, CHUNK)]], bufs.at[0], sem)
        pltpu.async_copy(table.at[itr.at[pl.ds(off, CHUNK)]], bufs.at[1], sem)

    def drain(bufs, sem):
        for k in range(2):
            pltpu.make_async_copy(table.at[pl.ds(0, CHUNK)],
                                  bufs.at[k], sem).wait()

    def blend(ci, bufs, rout):
        off = pl.multiple_of(ci * CHUNK, CHUNK)

        def group_body(g, gcarry):
            r0 = pl.multiple_of(g * LANES, LANES)
            wa = vtl[pl.ds(off + r0, LANES)]
            wb = vtr[pl.ds(off + r0, LANES)]
            wc = vbl[pl.ds(off + r0, LANES)]
            wd = vbr[pl.ds(off + r0, LANES)]
            mask_hi = jnp.int32(-65536)
            for k in range(LANES):
                r = r0 + k
                a, b, c, d = wa[k], wb[k], wc[k], wd[k]
                for g3 in range(C // 32):
                    xtl = bufs[0, r, pl.ds(g3 * LANES, LANES)]
                    xtr = bufs[0, r, pl.ds(NPACK + g3 * LANES, LANES)]
                    xbl = bufs[1, r, pl.ds(g3 * LANES, LANES)]
                    xbr = bufs[1, r, pl.ds(NPACK + g3 * LANES, LANES)]
                    acc_lo = (plsc.bitcast(xtl << 16, jnp.float32) * a
                              + plsc.bitcast(xtr << 16, jnp.float32) * b
                              + plsc.bitcast(xbl << 16, jnp.float32) * c
                              + plsc.bitcast(xbr << 16, jnp.float32) * d)
                    acc_hi = (plsc.bitcast(xtl & mask_hi, jnp.float32) * a
                              + plsc.bitcast(xtr & mask_hi, jnp.float32) * b
                              + plsc.bitcast(xbl & mask_hi, jnp.float32) * c
                              + plsc.bitcast(xbr & mask_hi, jnp.float32) * d)
                    rout[r, pl.ds(g3 * 32, LANES)] = acc_lo
                    rout[r, pl.ds(g3 * 32 + LANES, LANES)] = acc_hi
            return gcarry

        lax.fori_loop(0, CHUNK // LANES, group_body, 0, unroll=False)

    def out_fire(ci, rout):
        off = pl.multiple_of(ci * CHUNK, CHUNK)
        pltpu.async_copy(rout, out.at[pl.ds(base0 + off, CHUNK)], sem_out)

    def out_drain(rout):
        pltpu.make_async_copy(rout, out.at[pl.ds(base0, CHUNK)],
                              sem_out).wait()

    fire(0, *ring[0])

    def pair_body(g, carry):
        c0 = pl.multiple_of(g * 2, 2)
        fire(c0 + 1, *ring[1])
        drain(*ring[0])

        @pl.when(c0 > 0)
        def _():
            out_drain(rout0)

        blend(c0, ring[0][0], rout0)
        out_fire(c0, rout0)

        @pl.when(c0 + 2 < NCHUNK)
        def _():
            fire(c0 + 2, *ring[0])

        drain(*ring[1])

        @pl.when(c0 > 0)
        def _():
            out_drain(rout1)

        blend(c0 + 1, ring[1][0], rout1)
        out_fire(c0 + 1, rout1)
        return carry

    lax.fori_loop(0, NCHUNK // 2, pair_body, 0, unroll=False)
    out_drain(rout0)
    out_drain(rout1)


@functools.cache
def _make_sc_gather():
    return functools.partial(
        pl.kernel,
        out_type=jax.ShapeDtypeStruct((NPIX, C), jnp.float32),
        mesh=plsc.VectorSubcoreMesh(
            core_axis_name="c", subcore_axis_name="s",
            num_cores=NC, num_subcores=NS),
        scratch_types=(
            [pltpu.VMEM((ROWS_PER_W,), jnp.int32)] * 2
            + [pltpu.VMEM((ROWS_PER_W,), jnp.float32)] * 4
            + [pltpu.VMEM((2, CHUNK, TABLE_D), jnp.int32)] * 2
            + [pltpu.VMEM((CHUNK, C), jnp.float32)] * 2
            + [pltpu.SemaphoreType.DMA] * 3
        ),
        compiler_params=pltpu.CompilerParams(needs_layout_passes=False),
    )(_sc_body)


def kernel(image, boxes, box_ind):
    table = _make_table(image)
    ptl, ptr, wtl, wtr, wbl, wbr = _compute_indices(boxes, box_ind)
    flat = lambda a: a.reshape(NPIX)
    out_t = _make_sc_gather()(table, flat(ptl), flat(ptr), flat(wtl),
                              flat(wtr), flat(wbl), flat(wbr))
    return _to_nchw(out_t).reshape(N_BOX, C, CROP_H, CROP_W)
